# baseline (speedup 1.0000x reference)
import math

import jax
import jax.numpy as jnp
from jax import lax
from jax.experimental import pallas as pl
from jax.experimental.pallas import tpu as pltpu
from jax.experimental.pallas import tpu_sc as plsc

_N = 4096
_VOCAB = 32000
_IGNORE = 0
_EPS = 0.1
_U = _EPS / (_VOCAB - 2)
_C = (1.0 - _EPS) * math.log(1.0 - _EPS) + (_VOCAB - 2) * _U * math.log(_U)

_ROW_BLOCK = 128
_LANES = 16


def _tc_stream_kernel(tgt_ref, lp_ref, contrib_ref):
    tile = lp_ref[...]
    tgt = tgt_ref[0, 0, :]
    cols = jax.lax.broadcasted_iota(jnp.int32, tile.shape, 1)
    rowsum = jnp.sum(tile, axis=1)
    lp_t = jnp.sum(jnp.where(cols == tgt[:, None], tile, 0.0), axis=1)
    lp_0 = tile[:, 0]
    loss = _C + _U * lp_0 - (1.0 - _EPS - _U) * lp_t - _U * rowsum
    contrib_ref[...] = jnp.where(tgt == _IGNORE, 0.0, loss)


def _sc_reduce_body(contrib_hbm, out_hbm, contrib_v, red_v, out_v):
    wid = lax.axis_index("s") * 2 + lax.axis_index("c")
    n_chunks = contrib_v.shape[0] // _LANES

    @pl.when(wid == 0)
    def _():
        pltpu.sync_copy(contrib_hbm, contrib_v)

        def body(k, acc):
            return acc + contrib_v[pl.ds(k * _LANES, _LANES)]

        acc = lax.fori_loop(0, n_chunks, body,
                            jnp.zeros((_LANES,), jnp.float32))
        red_v[pl.ds(_LANES, _LANES)] = jnp.zeros((_LANES,), jnp.float32)
        for shift in (8, 4, 2, 1):
            red_v[pl.ds(0, _LANES)] = acc
            acc = acc + red_v[pl.ds(shift, _LANES)]
        out_v[...] = acc
        pltpu.sync_copy(out_v, out_hbm)


def _tc_stream_half(log_probs, tgt3, half_blocks, block_off):
    n, v = log_probs.shape
    r = _ROW_BLOCK
    return pl.pallas_call(
        _tc_stream_kernel,
        grid=(half_blocks,),
        in_specs=[
            pl.BlockSpec((1, 1, r), lambda i: (i + block_off, 0, 0)),
            pl.BlockSpec((r, v), lambda i: (i + block_off, 0)),
        ],
        out_specs=pl.BlockSpec((r,), lambda i: (i,)),
        out_shape=jax.ShapeDtypeStruct((half_blocks * r, ), jnp.float32),
    )(tgt3, log_probs)


def kernel(log_probs, targets):
    n, _ = log_probs.shape
    r = _ROW_BLOCK
    nb = n // r
    hb = nb // 2
    tgt3 = targets.reshape(nb, 1, r)
    mesh = plsc.VectorSubcoreMesh(core_axis_name="c", subcore_axis_name="s",
                                  num_cores=1)
    reduce = pl.kernel(
        _sc_reduce_body, mesh=mesh,
        out_type=jax.ShapeDtypeStruct((_LANES,), jnp.float32),
        scratch_types=[
            pltpu.VMEM((_N // 2,), jnp.float32),
            pltpu.VMEM((2 * _LANES,), jnp.float32),
            pltpu.VMEM((_LANES,), jnp.float32),
        ],
    )
    loss_a = reduce(_tc_stream_half(log_probs, tgt3, hb, 0))
    loss_b = reduce(_tc_stream_half(log_probs, tgt3, hb, hb))
    return loss_a[0] + loss_b[0]


# speedup vs baseline: 1.0361x; 1.0361x over previous
import math

import jax
import jax.numpy as jnp
from jax import lax
from jax.experimental import pallas as pl
from jax.experimental.pallas import tpu as pltpu
from jax.experimental.pallas import tpu_sc as plsc

_N = 4096
_VOCAB = 32000
_IGNORE = 0
_EPS = 0.1
_U = _EPS / (_VOCAB - 2)
_C = (1.0 - _EPS) * math.log(1.0 - _EPS) + (_VOCAB - 2) * _U * math.log(_U)

_ROW_BLOCK = 128
_LANES = 16


def _tc_stream_kernel(tgt_ref, lp_ref, contrib_ref):
    tile = lp_ref[...]
    tgt = tgt_ref[0, 0, :]
    cols = jax.lax.broadcasted_iota(jnp.int32, tile.shape, 1)
    rowsum = jnp.sum(tile, axis=1)
    lp_t = jnp.sum(jnp.where(cols == tgt[:, None], tile, 0.0), axis=1)
    lp_0 = tile[:, 0]
    loss = _C + _U * lp_0 - (1.0 - _EPS - _U) * lp_t - _U * rowsum
    contrib_ref[...] = jnp.where(tgt == _IGNORE, 0.0, loss)


def _sc_reduce_body(contrib_hbm, out_hbm, contrib_v, red_v, out_v):
    wid = lax.axis_index("s") * 2 + lax.axis_index("c")
    n_chunks = contrib_v.shape[0] // _LANES

    @pl.when(wid == 0)
    def _():
        pltpu.sync_copy(contrib_hbm, contrib_v)

        def body(k, acc):
            return acc + contrib_v[pl.ds(k * _LANES, _LANES)]

        acc = lax.fori_loop(0, n_chunks, body,
                            jnp.zeros((_LANES,), jnp.float32))
        red_v[pl.ds(_LANES, _LANES)] = jnp.zeros((_LANES,), jnp.float32)
        for shift in (8, 4, 2, 1):
            red_v[pl.ds(0, _LANES)] = acc
            acc = acc + red_v[pl.ds(shift, _LANES)]
        out_v[...] = acc
        pltpu.sync_copy(out_v, out_hbm)


def kernel(log_probs, targets):
    n, v = log_probs.shape
    r = _ROW_BLOCK
    nb = n // r
    contrib = pl.pallas_call(
        _tc_stream_kernel,
        grid=(nb,),
        in_specs=[
            pl.BlockSpec((1, 1, r), lambda i: (i, 0, 0)),
            pl.BlockSpec((r, v), lambda i: (i, 0)),
        ],
        out_specs=pl.BlockSpec((r,), lambda i: (i,)),
        out_shape=jax.ShapeDtypeStruct((n,), jnp.float32),
    )(targets.reshape(nb, 1, r), log_probs)

    mesh = plsc.VectorSubcoreMesh(core_axis_name="c", subcore_axis_name="s",
                                  num_cores=1)
    reduce = pl.kernel(
        _sc_reduce_body, mesh=mesh,
        out_type=jax.ShapeDtypeStruct((_LANES,), jnp.float32),
        scratch_types=[
            pltpu.VMEM((_N,), jnp.float32),
            pltpu.VMEM((2 * _LANES,), jnp.float32),
            pltpu.VMEM((_LANES,), jnp.float32),
        ],
    )
    loss16 = reduce(contrib)
    return loss16[0]
